# baseline (speedup 1.0000x reference)
import functools

import jax
import jax.numpy as jnp
from jax import lax
from jax.experimental import pallas as pl
from jax.experimental.pallas import tpu as pltpu
from jax.experimental.pallas import tpu_sc as plsc

_B = 8
_C = 96
_S = 224 * 224
_N = _C * _S
_K = int(round(0.1 * _N))

_NCORES = 2
_NSUB = 16
_CHUNK = _S // 4
_VREGS_PER_CHUNK = _CHUNK // 16

_MIN_I32 = -(2 ** 31)
_PASSES = ((21, 11), (10, 11), (0, 10))


def _sc_body(rows, group, x_hbm, bf_hbm, out_hbm, bf_v, stage0_v, stage1_v,
             hist_v, idx_v, tvec_v, merged_sh, sem0, sem1):
    rows_per_core = rows // _NCORES
    tiles_per_row = _NSUB // rows_per_core
    ch_per_tile = _C // tiles_per_row
    chunks_per_tile = ch_per_tile * 4
    zero_rows = rows_per_core * 2048 // _NSUB

    c = lax.axis_index("c")
    s = lax.axis_index("s")
    local_row = s // tiles_per_row
    grp = s % tiles_per_row
    grow = c * rows_per_core + local_row
    row = group * rows + grow
    lane = lax.iota(jnp.int32, 16)
    ones = jnp.ones((16,), jnp.int32)
    zeros16 = jnp.zeros((16,), jnp.int32)

    pltpu.sync_copy(bf_hbm, bf_v)

    def fill_idx(j, _):
        def fill_t(t, _):
            idx_v[j, pl.ds(t * 16, 16)] = (local_row * 2048 + j * 128
                                           + t * 16 + lane)
            return 0
        lax.fori_loop(0, 8, fill_t, 0)
        return 0
    lax.fori_loop(0, 16, fill_idx, 0)

    stages = (stage0_v, stage1_v)
    sems = (sem0, sem1)

    def chunk_src(i):
        ch = grp * ch_per_tile + i // 4
        q = i % 4
        return x_hbm.at[row, ch, pl.ds(q * _CHUNK, _CHUNK)]

    rem = jnp.int32(_K)
    prefix = jnp.int32(0)

    for shift, width in _PASSES:
        nbins = 1 << width
        first = shift == _PASSES[0][0]

        def zbody(i, _):
            hist_v[i, :] = zeros16
            return 0
        lax.fori_loop(0, 2048, zbody, 0)
        pltpu.sync_copy(hist_v.at[pl.ds(0, zero_rows), :],
                        merged_sh.at[pl.ds(s * zero_rows, zero_rows), :])
        plsc.subcore_barrier()

        shift_vec = jnp.full((16,), shift, jnp.int32)
        binmask_vec = jnp.full((16,), nbins - 1, jnp.int32)
        binsize_vec_u = jnp.full((16,), nbins, jnp.uint32)
        prefix_vec = jnp.full((16,), prefix << width, jnp.int32)

        pltpu.make_async_copy(chunk_src(0), stage0_v, sem0).start()
        pltpu.make_async_copy(chunk_src(1), stage1_v, sem1).start()

        def chunk_group(g, _):
            for b in range(2):
                i = 2 * g + b
                stage, sem = stages[b], sems[b]
                pltpu.make_async_copy(chunk_src(i), stage, sem).wait()
                ch = grp * ch_per_tile + i // 4
                bfv = bf_v[ch, :]

                @plsc.parallel_loop(0, _VREGS_PER_CHUNK, unroll=8)
                def _(j):
                    v = stage[pl.ds(j * 16, 16)]
                    boosted = v * bfv
                    bits = lax.bitcast_convert_type(boosted, jnp.int32)
                    neg = bits >> 31
                    key = bits ^ (neg | _MIN_I32)
                    if first:
                        bin_ = lax.shift_right_logical(key, shift_vec)
                        plsc.addupdate_scatter(hist_v, [bin_, lane], ones)
                    else:
                        rel = (lax.shift_right_logical(key, shift_vec)
                               - prefix_vec)
                        rel_u = lax.bitcast_convert_type(rel, jnp.uint32)
                        mk = rel_u < binsize_vec_u
                        bin_ = rel & binmask_vec
                        plsc.addupdate_scatter(hist_v, [bin_, lane], ones,
                                               mask=mk)

                @pl.when(i + 2 < chunks_per_tile)
                def _():
                    pltpu.make_async_copy(chunk_src(i + 2), stage, sem).start()
            return 0
        lax.fori_loop(0, chunks_per_tile // 2, chunk_group, 0)

        for j in range(16):
            pltpu.sync_copy(hist_v.at[pl.ds(j * 128, 128), :],
                            merged_sh.at[idx_v.at[j]], add=True)
        plsc.subcore_barrier()
        pltpu.sync_copy(merged_sh.at[pl.ds(local_row * 2048, 2048), :], hist_v)
        plsc.subcore_barrier()

        def scan_body(i, carry):
            cum, found, b_sel, rem_next = carry
            b = nbins - 1 - i
            cnt = jnp.sum(hist_v[b, :])
            cum_new = cum + cnt
            crossed = cum_new >= rem
            take = jnp.logical_and(crossed, jnp.logical_not(found))
            b_sel = jnp.where(take, b, b_sel)
            rem_next = jnp.where(take, rem - cum, rem_next)
            found = jnp.logical_or(found, crossed)
            return (cum_new, found, b_sel, rem_next)

        init = (jnp.int32(0), jnp.bool_(False), jnp.int32(0), rem)
        _, _, b_sel, rem = lax.fori_loop(0, nbins, scan_body, init)
        prefix = (prefix << width) | b_sel

    @pl.when(grp == 0)
    def _():
        tvec_v[...] = jnp.full((16,), prefix, jnp.int32)
        pltpu.sync_copy(tvec_v, out_hbm.at[grow])


@functools.lru_cache(maxsize=None)
def _sc_select_fn(rows, group):
    mesh = plsc.VectorSubcoreMesh(core_axis_name="c", subcore_axis_name="s")
    return pl.kernel(
        functools.partial(_sc_body, rows, group),
        out_type=jax.ShapeDtypeStruct((rows, 16), jnp.int32),
        mesh=mesh,
        scratch_types=[
            pltpu.VMEM((_C, 16), jnp.float32),
            pltpu.VMEM((_CHUNK,), jnp.float32),
            pltpu.VMEM((_CHUNK,), jnp.float32),
            pltpu.VMEM((2048, 16), jnp.int32),
            pltpu.VMEM((16, 128), jnp.int32),
            pltpu.VMEM((16,), jnp.int32),
            pltpu.VMEM_SHARED((rows // _NCORES * 2048, 16), jnp.int32),
            pltpu.SemaphoreType.DMA,
            pltpu.SemaphoreType.DMA,
        ],
        compiler_params=pltpu.CompilerParams(
            use_tc_tiling_on_sc=False,
            needs_layout_passes=False,
        ),
    )


def _apply_body(t_ref, bf_ref, x_ref, o_ref):
    xv = x_ref[...]
    boosted = xv * bf_ref[...]
    bits = lax.bitcast_convert_type(boosted, jnp.int32)
    neg = bits >> 31
    key = bits ^ (neg | _MIN_I32)
    key_u = lax.bitcast_convert_type(key, jnp.uint32)
    t_u = lax.bitcast_convert_type(t_ref[0, 0, 0], jnp.uint32)
    o_ref[...] = jnp.where(key_u >= t_u, xv, jnp.float32(0.0))


def _apply_body_acc(t_ref, bf_ref, x_ref, acc_ref, o_ref):
    del acc_ref
    _apply_body(t_ref, bf_ref, x_ref, o_ref)


def _tc_apply(x3, bf3, thresholds, rows, group, acc):
    CB = 16
    t_spec = pl.BlockSpec((1, 1, 16), lambda b, j: (b, 0, 0),
                          memory_space=pltpu.SMEM)
    bf_spec = pl.BlockSpec((1, CB, 1), lambda b, j: (0, j, 0))
    x_spec = pl.BlockSpec((1, CB, _S), lambda b, j: (group * rows + b, j, 0))
    args = [thresholds.reshape(rows, 1, 16), bf3, x3]
    in_specs = [t_spec, bf_spec, x_spec]
    kwargs = {}
    body = _apply_body
    if acc is not None:
        in_specs.append(pl.BlockSpec(memory_space=pl.ANY))
        args.append(acc)
        kwargs["input_output_aliases"] = {3: 0}
        body = _apply_body_acc
    return pl.pallas_call(
        body,
        grid=(rows, _C // CB),
        in_specs=in_specs,
        out_specs=pl.BlockSpec((1, CB, _S),
                               lambda b, j: (group * rows + b, j, 0)),
        out_shape=jax.ShapeDtypeStruct((_B, _C, _S), jnp.float32),
        **kwargs,
    )(*args)


def kernel(x, duty_cycles):
    B, C, H, W = x.shape
    bf = jnp.exp(jnp.float32(_K / _N) - duty_cycles.reshape(C))
    bf_pad = jnp.broadcast_to(bf.reshape(C, 1), (C, 16))
    bf3 = bf.reshape(1, C, 1)
    x3 = x.reshape(B, C, _S)

    rows = 4
    ngroups = B // rows
    thresholds = [_sc_select_fn(rows, g)(x3, bf_pad) for g in range(ngroups)]
    acc = None
    for g in range(ngroups):
        acc = _tc_apply(x3, bf3, thresholds[g], rows, g, acc)
    return acc.reshape(B, C, H, W)


# speedup vs baseline: 1.0660x; 1.0660x over previous
import functools

import jax
import jax.numpy as jnp
from jax import lax
from jax.experimental import pallas as pl
from jax.experimental.pallas import tpu as pltpu
from jax.experimental.pallas import tpu_sc as plsc

_B = 8
_C = 96
_S = 224 * 224
_N = _C * _S
_K = int(round(0.1 * _N))

_NCORES = 2
_NSUB = 16
_CHUNK = _S // 4
_VREGS_PER_CHUNK = _CHUNK // 16

_MIN_I32 = -(2 ** 31)
_PASSES = ((21, 11), (10, 11), (0, 10))


def _sc_body(rows, group, x_hbm, bf_hbm, out_hbm, bf_v, stage0_v, stage1_v,
             hist_v, idx_v, tvec_v, merged_sh, sem0, sem1):
    rows_per_core = rows // _NCORES
    tiles_per_row = _NSUB // rows_per_core
    ch_per_tile = _C // tiles_per_row
    chunks_per_tile = ch_per_tile * 4
    zero_rows = rows_per_core * 2048 // _NSUB

    c = lax.axis_index("c")
    s = lax.axis_index("s")
    local_row = s // tiles_per_row
    grp = s % tiles_per_row
    grow = c * rows_per_core + local_row
    row = group * rows + grow
    lane = lax.iota(jnp.int32, 16)
    ones = jnp.ones((16,), jnp.int32)
    zeros16 = jnp.zeros((16,), jnp.int32)

    pltpu.sync_copy(bf_hbm, bf_v)

    def fill_idx(j, _):
        def fill_t(t, _):
            idx_v[j, pl.ds(t * 16, 16)] = (local_row * 2048 + j * 128
                                           + t * 16 + lane)
            return 0
        lax.fori_loop(0, 8, fill_t, 0)
        return 0
    lax.fori_loop(0, 16, fill_idx, 0)

    stages = (stage0_v, stage1_v)
    sems = (sem0, sem1)

    def chunk_src(i):
        ch = grp * ch_per_tile + i // 4
        q = i % 4
        return x_hbm.at[row, ch, pl.ds(q * _CHUNK, _CHUNK)]

    rem = jnp.int32(_K)
    prefix = jnp.int32(0)

    for shift, width in _PASSES:
        nbins = 1 << width
        first = shift == _PASSES[0][0]

        def zbody(i, _):
            hist_v[i, :] = zeros16
            return 0
        lax.fori_loop(0, 2048, zbody, 0)
        pltpu.sync_copy(hist_v.at[pl.ds(0, zero_rows), :],
                        merged_sh.at[pl.ds(s * zero_rows, zero_rows), :])
        plsc.subcore_barrier()

        shift_vec = jnp.full((16,), shift, jnp.int32)
        binmask_vec = jnp.full((16,), nbins - 1, jnp.int32)
        binsize_vec_u = jnp.full((16,), nbins, jnp.uint32)
        prefix_vec = jnp.full((16,), prefix << width, jnp.int32)

        pltpu.make_async_copy(chunk_src(0), stage0_v, sem0).start()
        pltpu.make_async_copy(chunk_src(1), stage1_v, sem1).start()

        def chunk_group(g, _):
            for b in range(2):
                i = 2 * g + b
                stage, sem = stages[b], sems[b]
                pltpu.make_async_copy(chunk_src(i), stage, sem).wait()
                ch = grp * ch_per_tile + i // 4
                bfv = bf_v[ch, :]

                @plsc.parallel_loop(0, _VREGS_PER_CHUNK, unroll=14)
                def _(j):
                    v = stage[pl.ds(j * 16, 16)]
                    boosted = v * bfv
                    bits = lax.bitcast_convert_type(boosted, jnp.int32)
                    neg = bits >> 31
                    key = bits ^ (neg | _MIN_I32)
                    if first:
                        bin_ = lax.shift_right_logical(key, shift_vec)
                        plsc.addupdate_scatter(hist_v, [bin_, lane], ones)
                    else:
                        rel = (lax.shift_right_logical(key, shift_vec)
                               - prefix_vec)
                        rel_u = lax.bitcast_convert_type(rel, jnp.uint32)
                        mk = rel_u < binsize_vec_u
                        bin_ = rel & binmask_vec
                        plsc.addupdate_scatter(hist_v, [bin_, lane], ones,
                                               mask=mk)

                @pl.when(i + 2 < chunks_per_tile)
                def _():
                    pltpu.make_async_copy(chunk_src(i + 2), stage, sem).start()
            return 0
        lax.fori_loop(0, chunks_per_tile // 2, chunk_group, 0)

        for j in range(16):
            pltpu.sync_copy(hist_v.at[pl.ds(j * 128, 128), :],
                            merged_sh.at[idx_v.at[j]], add=True)
        plsc.subcore_barrier()
        pltpu.sync_copy(merged_sh.at[pl.ds(local_row * 2048, 2048), :], hist_v)
        plsc.subcore_barrier()

        def scan_body(i, carry):
            cum, found, b_sel, rem_next = carry
            b = nbins - 1 - i
            cnt = jnp.sum(hist_v[b, :])
            cum_new = cum + cnt
            crossed = cum_new >= rem
            take = jnp.logical_and(crossed, jnp.logical_not(found))
            b_sel = jnp.where(take, b, b_sel)
            rem_next = jnp.where(take, rem - cum, rem_next)
            found = jnp.logical_or(found, crossed)
            return (cum_new, found, b_sel, rem_next)

        init = (jnp.int32(0), jnp.bool_(False), jnp.int32(0), rem)
        _, _, b_sel, rem = lax.fori_loop(0, nbins, scan_body, init)
        prefix = (prefix << width) | b_sel

    @pl.when(grp == 0)
    def _():
        tvec_v[...] = jnp.full((16,), prefix, jnp.int32)
        pltpu.sync_copy(tvec_v, out_hbm.at[grow])


@functools.lru_cache(maxsize=None)
def _sc_select_fn(rows, group):
    mesh = plsc.VectorSubcoreMesh(core_axis_name="c", subcore_axis_name="s")
    return pl.kernel(
        functools.partial(_sc_body, rows, group),
        out_type=jax.ShapeDtypeStruct((rows, 16), jnp.int32),
        mesh=mesh,
        scratch_types=[
            pltpu.VMEM((_C, 16), jnp.float32),
            pltpu.VMEM((_CHUNK,), jnp.float32),
            pltpu.VMEM((_CHUNK,), jnp.float32),
            pltpu.VMEM((2048, 16), jnp.int32),
            pltpu.VMEM((16, 128), jnp.int32),
            pltpu.VMEM((16,), jnp.int32),
            pltpu.VMEM_SHARED((rows // _NCORES * 2048, 16), jnp.int32),
            pltpu.SemaphoreType.DMA,
            pltpu.SemaphoreType.DMA,
        ],
        compiler_params=pltpu.CompilerParams(
            use_tc_tiling_on_sc=False,
            needs_layout_passes=False,
        ),
    )


def _apply_body(t_ref, bf_ref, x_ref, o_ref):
    xv = x_ref[...]
    boosted = xv * bf_ref[...]
    bits = lax.bitcast_convert_type(boosted, jnp.int32)
    neg = bits >> 31
    key = bits ^ (neg | _MIN_I32)
    key_u = lax.bitcast_convert_type(key, jnp.uint32)
    t_u = lax.bitcast_convert_type(t_ref[0, 0, 0], jnp.uint32)
    o_ref[...] = jnp.where(key_u >= t_u, xv, jnp.float32(0.0))


def _apply_body_acc(t_ref, bf_ref, x_ref, acc_ref, o_ref):
    del acc_ref
    _apply_body(t_ref, bf_ref, x_ref, o_ref)


def _tc_apply(x3, bf3, thresholds, rows, group, acc):
    CB = 16
    t_spec = pl.BlockSpec((1, 1, 16), lambda b, j: (b, 0, 0),
                          memory_space=pltpu.SMEM)
    bf_spec = pl.BlockSpec((1, CB, 1), lambda b, j: (0, j, 0))
    x_spec = pl.BlockSpec((1, CB, _S), lambda b, j: (group * rows + b, j, 0))
    args = [thresholds.reshape(rows, 1, 16), bf3, x3]
    in_specs = [t_spec, bf_spec, x_spec]
    kwargs = {}
    body = _apply_body
    if acc is not None:
        in_specs.append(pl.BlockSpec(memory_space=pl.ANY))
        args.append(acc)
        kwargs["input_output_aliases"] = {3: 0}
        body = _apply_body_acc
    return pl.pallas_call(
        body,
        grid=(rows, _C // CB),
        in_specs=in_specs,
        out_specs=pl.BlockSpec((1, CB, _S),
                               lambda b, j: (group * rows + b, j, 0)),
        out_shape=jax.ShapeDtypeStruct((_B, _C, _S), jnp.float32),
        **kwargs,
    )(*args)


def kernel(x, duty_cycles):
    B, C, H, W = x.shape
    bf = jnp.exp(jnp.float32(_K / _N) - duty_cycles.reshape(C))
    bf_pad = jnp.broadcast_to(bf.reshape(C, 1), (C, 16))
    bf3 = bf.reshape(1, C, 1)
    x3 = x.reshape(B, C, _S)

    thresholds = _sc_select_fn(_B, 0)(x3, bf_pad)
    out = _tc_apply(x3, bf3, thresholds, _B, 0, None)
    return out.reshape(B, C, H, W)


# speedup vs baseline: 1.0761x; 1.0094x over previous
import functools

import jax
import jax.numpy as jnp
from jax import lax
from jax.experimental import pallas as pl
from jax.experimental.pallas import tpu as pltpu
from jax.experimental.pallas import tpu_sc as plsc

_B = 8
_C = 96
_S = 224 * 224
_N = _C * _S
_K = int(round(0.1 * _N))

_NCORES = 2
_NSUB = 16
_CHUNK = _S // 4
_VREGS_PER_CHUNK = _CHUNK // 16

_MIN_I32 = -(2 ** 31)
_PASSES = ((21, 11), (10, 11), (0, 10))


def _sc_body(rows, group, x_hbm, bf_hbm, out_hbm, bf_v, stage0_v, stage1_v,
             hist_v, idx_v, tvec_v, merged_sh, sem0, sem1):
    rows_per_core = rows // _NCORES
    tiles_per_row = _NSUB // rows_per_core
    ch_per_tile = _C // tiles_per_row
    chunks_per_tile = ch_per_tile * 4
    zero_rows = rows_per_core * 2048 // _NSUB

    c = lax.axis_index("c")
    s = lax.axis_index("s")
    local_row = s // tiles_per_row
    grp = s % tiles_per_row
    grow = c * rows_per_core + local_row
    row = group * rows + grow
    lane = lax.iota(jnp.int32, 16)
    ones = jnp.ones((16,), jnp.int32)
    zeros16 = jnp.zeros((16,), jnp.int32)

    pltpu.sync_copy(bf_hbm, bf_v)

    def fill_idx(j, _):
        def fill_t(t, _):
            idx_v[j, pl.ds(t * 16, 16)] = (local_row * 2048 + j * 128
                                           + t * 16 + lane)
            return 0
        lax.fori_loop(0, 8, fill_t, 0)
        return 0
    lax.fori_loop(0, 16, fill_idx, 0)

    stages = (stage0_v, stage1_v)
    sems = (sem0, sem1)

    def chunk_src(i):
        ch = grp * ch_per_tile + i // 4
        q = i % 4
        return x_hbm.at[row, ch, pl.ds(q * _CHUNK, _CHUNK)]

    rem = jnp.int32(_K)
    prefix = jnp.int32(0)

    for shift, width in _PASSES:
        nbins = 1 << width
        first = shift == _PASSES[0][0]

        def zbody(i, _):
            hist_v[i, :] = zeros16
            return 0
        lax.fori_loop(0, 2048, zbody, 0)
        pltpu.sync_copy(hist_v.at[pl.ds(0, zero_rows), :],
                        merged_sh.at[pl.ds(s * zero_rows, zero_rows), :])
        plsc.subcore_barrier()

        shift_vec = jnp.full((16,), shift, jnp.int32)
        binmask_vec = jnp.full((16,), nbins - 1, jnp.int32)
        binsize_vec_u = jnp.full((16,), nbins, jnp.uint32)
        prefix_vec = jnp.full((16,), prefix << width, jnp.int32)

        pltpu.make_async_copy(chunk_src(0), stage0_v, sem0).start()
        pltpu.make_async_copy(chunk_src(1), stage1_v, sem1).start()

        def chunk_group(g, _):
            for b in range(2):
                i = 2 * g + b
                stage, sem = stages[b], sems[b]
                pltpu.make_async_copy(chunk_src(i), stage, sem).wait()
                ch = grp * ch_per_tile + i // 4
                bfv = bf_v[ch, :]

                @plsc.parallel_loop(0, _VREGS_PER_CHUNK, unroll=8)
                def _(j):
                    v = stage[pl.ds(j * 16, 16)]
                    boosted = v * bfv
                    bits = lax.bitcast_convert_type(boosted, jnp.int32)
                    neg = bits >> 31
                    key = bits ^ (neg | _MIN_I32)
                    if first:
                        bin_ = lax.shift_right_logical(key, shift_vec)
                        plsc.addupdate_scatter(hist_v, [bin_, lane], ones)
                    else:
                        rel = (lax.shift_right_logical(key, shift_vec)
                               - prefix_vec)
                        rel_u = lax.bitcast_convert_type(rel, jnp.uint32)
                        mk = rel_u < binsize_vec_u
                        bin_ = rel & binmask_vec
                        plsc.addupdate_scatter(hist_v, [bin_, lane], ones,
                                               mask=mk)

                @pl.when(i + 2 < chunks_per_tile)
                def _():
                    pltpu.make_async_copy(chunk_src(i + 2), stage, sem).start()
            return 0
        lax.fori_loop(0, chunks_per_tile // 2, chunk_group, 0)

        for j in range(16):
            pltpu.sync_copy(hist_v.at[pl.ds(j * 128, 128), :],
                            merged_sh.at[idx_v.at[j]], add=True)
        plsc.subcore_barrier()
        pltpu.sync_copy(merged_sh.at[pl.ds(local_row * 2048, 2048), :], hist_v)
        plsc.subcore_barrier()

        def scan_body(i, carry):
            cum, found, b_sel, rem_next = carry
            b = nbins - 1 - i
            cnt = jnp.sum(hist_v[b, :])
            cum_new = cum + cnt
            crossed = cum_new >= rem
            take = jnp.logical_and(crossed, jnp.logical_not(found))
            b_sel = jnp.where(take, b, b_sel)
            rem_next = jnp.where(take, rem - cum, rem_next)
            found = jnp.logical_or(found, crossed)
            return (cum_new, found, b_sel, rem_next)

        init = (jnp.int32(0), jnp.bool_(False), jnp.int32(0), rem)
        _, _, b_sel, rem = lax.fori_loop(0, nbins, scan_body, init)
        prefix = (prefix << width) | b_sel

    @pl.when(grp == 0)
    def _():
        tvec_v[...] = jnp.full((16,), prefix, jnp.int32)
        pltpu.sync_copy(tvec_v, out_hbm.at[grow])


@functools.lru_cache(maxsize=None)
def _sc_select_fn(rows, group):
    mesh = plsc.VectorSubcoreMesh(core_axis_name="c", subcore_axis_name="s")
    return pl.kernel(
        functools.partial(_sc_body, rows, group),
        out_type=jax.ShapeDtypeStruct((rows, 16), jnp.int32),
        mesh=mesh,
        scratch_types=[
            pltpu.VMEM((_C, 16), jnp.float32),
            pltpu.VMEM((_CHUNK,), jnp.float32),
            pltpu.VMEM((_CHUNK,), jnp.float32),
            pltpu.VMEM((2048, 16), jnp.int32),
            pltpu.VMEM((16, 128), jnp.int32),
            pltpu.VMEM((16,), jnp.int32),
            pltpu.VMEM_SHARED((rows // _NCORES * 2048, 16), jnp.int32),
            pltpu.SemaphoreType.DMA,
            pltpu.SemaphoreType.DMA,
        ],
        compiler_params=pltpu.CompilerParams(
            use_tc_tiling_on_sc=False,
            needs_layout_passes=False,
        ),
    )


def _apply_body(t_ref, bf_ref, x_ref, o_ref):
    xv = x_ref[...]
    boosted = xv * bf_ref[...]
    bits = lax.bitcast_convert_type(boosted, jnp.int32)
    neg = bits >> 31
    key = bits ^ (neg | _MIN_I32)
    key_u = lax.bitcast_convert_type(key, jnp.uint32)
    t_u = lax.bitcast_convert_type(t_ref[0, 0, 0], jnp.uint32)
    o_ref[...] = jnp.where(key_u >= t_u, xv, jnp.float32(0.0))


def _tc_apply(x3, bf3, thresholds, rows):
    NS = 4
    SB = _S // NS
    return pl.pallas_call(
        _apply_body,
        grid=(rows, NS),
        in_specs=[
            pl.BlockSpec((1, 1, 16), lambda b, j: (b, 0, 0),
                         memory_space=pltpu.SMEM),
            pl.BlockSpec((1, _C, 1), lambda b, j: (0, 0, 0)),
            pl.BlockSpec((1, _C, SB), lambda b, j: (b, 0, j)),
        ],
        out_specs=pl.BlockSpec((1, _C, SB), lambda b, j: (b, 0, j)),
        out_shape=jax.ShapeDtypeStruct((rows, _C, _S), jnp.float32),
    )(thresholds.reshape(rows, 1, 16), bf3, x3)


def kernel(x, duty_cycles):
    B, C, H, W = x.shape
    bf = jnp.exp(jnp.float32(_K / _N) - duty_cycles.reshape(C))
    bf_pad = jnp.broadcast_to(bf.reshape(C, 1), (C, 16))
    bf3 = bf.reshape(1, C, 1)
    x3 = x.reshape(B, C, _S)

    thresholds = _sc_select_fn(_B, 0)(x3, bf_pad)
    out = _tc_apply(x3, bf3, thresholds, _B)
    return out.reshape(B, C, H, W)
